# baseline (speedup 1.0000x reference)
import functools

import jax
import jax.numpy as jnp
from jax import lax
from jax.experimental import pallas as pl
from jax.experimental.pallas import tpu as pltpu
from jax.experimental.pallas import tpu_sc as plsc

NU = 5000
NI = 5000
D = 128
E = 160000

NCORES = 2
NSUB = 16
NW = NCORES * NSUB
CH = 128
NCH = 40
EPAD = NW * NCH * CH
NP = NU + 8

_mesh = lambda: plsc.VectorSubcoreMesh(core_axis_name="c", subcore_axis_name="s")


def _zero_vmem(ref, rows, width):
    @pl.loop(0, rows)
    def _(i):
        for k in range(width // 16):
            ref[i, pl.ds(16 * k, 16)] = jnp.zeros((16,), jnp.float32)


def _fill_ones(ref, rows, width):
    @pl.loop(0, rows)
    def _(i):
        for k in range(width // 16):
            ref[i, pl.ds(16 * k, 16)] = jnp.ones((16,), jnp.float32)


def _rows_sweep(sub, fn):
    fn(pl.multiple_of(sub * 312, 8), 312)
    @pl.when(sub == 0)
    def _():
        fn(4992, 8)


def _copy_idx_row(src2d, j, dst1d):
    for i in range(CH // 16):
        dst1d[pl.ds(16 * i, 16)] = src2d[j, pl.ds(16 * i, 16)]


def _zero_rows(sub, acc, zhbm):
    _rows_sweep(sub, lambda off, n: pltpu.sync_copy(
        zhbm.at[pl.ds(off, n)], acc.at[pl.ds(off, n)]))


def _sc_degrees(ss_s, ss_d, ui_u, ui_i, zd, out_hbm,
                acc_a, acc_b, ix0, ix1, ix2, ix3, sidx_a, sidx_b,
                ones_v, sem_sa, sem_sb):
    core = lax.axis_index("c")
    sub = lax.axis_index("s")
    g = core * NSUB + sub

    _fill_ones(ones_v, CH, D)
    for edges, ix in zip((ss_s, ss_d, ui_u, ui_i), (ix0, ix1, ix2, ix3)):
        pltpu.sync_copy(edges.at[g], ix)

    for p, (ixa, ixb) in enumerate(((ix0, ix1), (ix2, ix3))):
        _zero_rows(sub, acc_a, zd)
        _zero_rows(sub, acc_b, zd)
        plsc.subcore_barrier()

        @pl.loop(0, NCH)
        def _(j):
            _copy_idx_row(ixa, j, sidx_a)
            _copy_idx_row(ixb, j, sidx_b)
            sa = pltpu.async_copy(ones_v, acc_a.at[sidx_a], sem_sa, add=True)
            sb = pltpu.async_copy(ones_v, acc_b.at[sidx_b], sem_sb, add=True)
            sa.wait()
            sb.wait()

        plsc.subcore_barrier()
        for base_off, acc in ((0, acc_a), (NU, acc_b)):
            _rows_sweep(sub, lambda off, n, a=acc, o=base_off, pp=p:
                        pltpu.sync_copy(
                            a.at[pl.ds(off, n)],
                            out_hbm.at[core, pp,
                                       pl.ds(pl.multiple_of(o + off, 8), n)]))


def _degrees(ss_s, ss_d, ui_u, ui_i, zd):
    fn = pl.kernel(
        _sc_degrees,
        out_type=jax.ShapeDtypeStruct((NCORES, 2, NU + NI, D), jnp.float32),
        mesh=_mesh(),
        scratch_types=[
            pltpu.VMEM_SHARED((NP, D), jnp.float32),
            pltpu.VMEM_SHARED((NP, D), jnp.float32),
            pltpu.VMEM((NCH, CH), jnp.int32),
            pltpu.VMEM((NCH, CH), jnp.int32),
            pltpu.VMEM((NCH, CH), jnp.int32),
            pltpu.VMEM((NCH, CH), jnp.int32),
            pltpu.VMEM((CH,), jnp.int32),
            pltpu.VMEM((CH,), jnp.int32),
            pltpu.VMEM((CH, D), jnp.float32),
            pltpu.SemaphoreType.DMA,
            pltpu.SemaphoreType.DMA,
        ],
    )
    return fn(ss_s, ss_d, ui_u, ui_i, zd)


def _sc_spmm_ss(x0, x1, x2, ss_s, ss_d, zd, out_hbm,
                acc, idx_s, idx_d, sidx, rows_a, rows_b, rows_c, rows_d,
                sem_ga, sem_gb, sem_gc, sem_gd):
    core = lax.axis_index("c")
    sub = lax.axis_index("s")
    g = core * NSUB + sub

    pltpu.sync_copy(ss_s.at[g], idx_s)
    pltpu.sync_copy(ss_d.at[g], idx_d)

    rows = (rows_a, rows_b, rows_c, rows_d)
    gsems = (sem_ga, sem_gb, sem_gc, sem_gd)
    for c, x in enumerate((x0, x1, x2)):
        _zero_rows(sub, acc, zd)
        plsc.subcore_barrier()

        @pl.loop(0, NCH // 4)
        def _(j):
            c0 = j * 4
            dg = [pltpu.async_copy(x.at[idx_s.at[c0 + k]], rows[k], gsems[k])
                  for k in range(4)]
            for k in range(4):
                dg[k].wait()
                _copy_idx_row(idx_d, c0 + k, sidx)
                pltpu.sync_copy(rows[k], acc.at[sidx], add=True)

        plsc.subcore_barrier()
        _rows_sweep(sub, lambda off, n, ci=c: pltpu.sync_copy(
            acc.at[pl.ds(off, n)], out_hbm.at[core, ci, pl.ds(off, n)]))


def _spmm_ss(x0, x1, x2, ss_s, ss_d, zd):
    fn = pl.kernel(
        _sc_spmm_ss,
        out_type=jax.ShapeDtypeStruct((NCORES, 3, NU, D), jnp.float32),
        mesh=_mesh(),
        scratch_types=[
            pltpu.VMEM_SHARED((NP, D), jnp.float32),
            pltpu.VMEM((NCH, CH), jnp.int32),
            pltpu.VMEM((NCH, CH), jnp.int32),
            pltpu.VMEM((CH,), jnp.int32),
            pltpu.VMEM((CH, D), jnp.float32),
            pltpu.VMEM((CH, D), jnp.float32),
            pltpu.VMEM((CH, D), jnp.float32),
            pltpu.VMEM((CH, D), jnp.float32),
        ] + [pltpu.SemaphoreType.DMA] * 4,
    )
    return fn(x0, x1, x2, ss_s, ss_d, zd)


def _sc_spmm_ui(xu, xi, ui_u, ui_i, zd, out_hbm,
                accu, acci, idx_u, idx_i, sidx_u, sidx_i, rows_a, rows_b,
                sem_ga, sem_gb, sem_sa, sem_sb):
    core = lax.axis_index("c")
    sub = lax.axis_index("s")
    g = core * NSUB + sub

    pltpu.sync_copy(ui_u.at[g], idx_u)
    pltpu.sync_copy(ui_i.at[g], idx_i)
    for acc in (accu, acci):
        _zero_rows(sub, acc, zd)
    plsc.subcore_barrier()

    @pl.loop(0, NCH)
    def _(j):
        du = pltpu.async_copy(xu.at[idx_u.at[j]], rows_a, sem_ga)
        di = pltpu.async_copy(xi.at[idx_i.at[j]], rows_b, sem_gb)
        _copy_idx_row(idx_i, j, sidx_i)
        _copy_idx_row(idx_u, j, sidx_u)
        du.wait()
        sa = pltpu.async_copy(rows_a, acci.at[sidx_i], sem_sa, add=True)
        di.wait()
        sb = pltpu.async_copy(rows_b, accu.at[sidx_u], sem_sb, add=True)
        sa.wait()
        sb.wait()

    plsc.subcore_barrier()
    for base_off, acc in ((0, accu), (NU, acci)):
        _rows_sweep(sub, lambda off, n, a=acc, o=base_off: pltpu.sync_copy(
            a.at[pl.ds(off, n)],
            out_hbm.at[core, pl.ds(pl.multiple_of(o + off, 8), n)]))


def _spmm_ui(xu, xi, ui_u, ui_i, zd):
    fn = pl.kernel(
        _sc_spmm_ui,
        out_type=jax.ShapeDtypeStruct((NCORES, NU + NI, D), jnp.float32),
        mesh=_mesh(),
        scratch_types=[
            pltpu.VMEM_SHARED((NP, D), jnp.float32),
            pltpu.VMEM_SHARED((NP, D), jnp.float32),
            pltpu.VMEM((NCH, CH), jnp.int32),
            pltpu.VMEM((NCH, CH), jnp.int32),
            pltpu.VMEM((CH,), jnp.int32),
            pltpu.VMEM((CH,), jnp.int32),
            pltpu.VMEM((CH, D), jnp.float32),
            pltpu.VMEM((CH, D), jnp.float32),
            pltpu.SemaphoreType.DMA,
            pltpu.SemaphoreType.DMA,
            pltpu.SemaphoreType.DMA,
            pltpu.SemaphoreType.DMA,
        ],
    )
    return fn(xu, xi, ui_u, ui_i, zd)


def _tc_prep_body(u_ref, gw_ref, gb_ref, degp_ref,
                  x0_ref, x1_ref, x2_ref, acc_ref, r_ref):
    u = u_ref[...]
    degp = degp_ref[..., 0:1]
    dsum = degp[0] + degp[1]
    d = jnp.stack([dsum[0, 0:NU], dsum[0, NU:],
                   dsum[1, 0:NU], dsum[1, NU:]], axis=0)
    r = lax.rsqrt(d + 1.0)
    r_ref[...] = r
    rs = r[0]
    xr = (x0_ref, x1_ref, x2_ref)
    for c in range(3):
        gate = jax.nn.sigmoid(
            jnp.dot(u, gw_ref[c], preferred_element_type=jnp.float32)
            + gb_ref[c][None, :])
        cur = u * gate
        acc_ref[c] = cur
        xr[c][0:NU] = cur * rs
        xr[c][NU:] = jnp.zeros((NP - NU, D), jnp.float32)


def _tc_prep(user_emb, gate_W, gate_b, degp):
    sd = jax.ShapeDtypeStruct
    return pl.pallas_call(
        _tc_prep_body,
        out_shape=(sd((NP, D), jnp.float32), sd((NP, D), jnp.float32),
                   sd((NP, D), jnp.float32), sd((3, NU, D), jnp.float32),
                   sd((4, NU, 1), jnp.float32)),
    )(user_emb, gate_W, gate_b, degp)


def _tc_comb_ss_body(need_x, p_ref, accin_ref, r_ref, *outs):
    r = r_ref[...]
    cur = (p_ref[0] + p_ref[1]) * r[1][None]
    acc = accin_ref[...] + cur
    if need_x:
        acc_ref, x0_ref, x1_ref, x2_ref = outs
        rs = r[0]
        zpad = jnp.zeros((NP - NU, D), jnp.float32)
        for c, xref in enumerate((x0_ref, x1_ref, x2_ref)):
            xref[0:NU] = cur[c] * rs
            xref[NU:] = zpad
    else:
        (acc_ref,) = outs
    acc_ref[...] = acc


def _tc_comb_ss(partial, acc_in, r, need_x):
    sd = jax.ShapeDtypeStruct
    outs = (sd((3, NU, D), jnp.float32),)
    if need_x:
        outs = outs + (sd((NP, D), jnp.float32),) * 3
    return pl.pallas_call(
        functools.partial(_tc_comb_ss_body, need_x),
        out_shape=outs,
    )(partial, acc_in, r)


def _tc_fuse_body(acc_ref, r_ref, item_ref, attn_ref,
                  xu_ref, xi_ref, accui_ref):
    r = r_ref[...]
    item = item_ref[...]
    ch = acc_ref[...] * (1.0 / 3.0)
    t = jnp.tanh(ch)
    s = jnp.sum(t * attn_ref[...][None, None, :], axis=-1, keepdims=True)
    m = jnp.max(s, axis=0, keepdims=True)
    e = jnp.exp(s - m)
    w = e / jnp.sum(e, axis=0, keepdims=True)
    uf = jnp.sum(w * ch, axis=0)
    accui_ref[0:NU] = uf
    accui_ref[NU:] = item
    zpad = jnp.zeros((NP - NU, D), jnp.float32)
    xu_ref[0:NU] = uf * r[2]
    xu_ref[NU:] = zpad
    xi_ref[0:NI] = item * r[3]
    xi_ref[NI:] = zpad


def _tc_fuse(acc, r, item_emb, attn_W):
    sd = jax.ShapeDtypeStruct
    return pl.pallas_call(
        _tc_fuse_body,
        out_shape=(sd((NP, D), jnp.float32), sd((NP, D), jnp.float32),
                   sd((NU + NI, D), jnp.float32)),
    )(acc, r, item_emb, attn_W)


def _tc_comb_ui_body(final, p_ref, accin_ref, r_ref, *outs):
    r = r_ref[...]
    pp = p_ref[0] + p_ref[1]
    cur_u = pp[0:NU] * r[2]
    cur_i = pp[NU:] * r[3]
    acc_u = accin_ref[0:NU] + cur_u
    acc_i = accin_ref[NU:] + cur_i
    if final:
        (out_ref,) = outs
        out_ref[0:NU] = acc_u * (1.0 / 3.0)
        out_ref[NU:] = acc_i * (1.0 / 3.0)
    else:
        acc_ref, xu_ref, xi_ref = outs
        acc_ref[0:NU] = acc_u
        acc_ref[NU:] = acc_i
        zpad = jnp.zeros((NP - NU, D), jnp.float32)
        xu_ref[0:NU] = cur_u * r[2]
        xu_ref[NU:] = zpad
        xi_ref[0:NI] = cur_i * r[3]
        xi_ref[NI:] = zpad


def _tc_comb_ui(partial, acc_in, r, final):
    sd = jax.ShapeDtypeStruct
    if final:
        outs = sd((NU + NI, D), jnp.float32)
    else:
        outs = (sd((NU + NI, D), jnp.float32), sd((NP, D), jnp.float32),
                sd((NP, D), jnp.float32))
    return pl.pallas_call(
        functools.partial(_tc_comb_ui_body, final),
        out_shape=outs,
    )(partial, acc_in, r)


def kernel(user_emb, item_emb, gate_W, gate_b, attn_W,
           ui_edge_index, ss_edge_index):
    pad = jnp.full((EPAD - E,), NU, jnp.int32)
    def _shape_edges(v):
        return jnp.concatenate([v, pad]).reshape(NW, NCH, CH)
    ss_s = _shape_edges(ss_edge_index[0])
    ss_d = _shape_edges(ss_edge_index[1])
    ui_u = _shape_edges(ui_edge_index[0])
    ui_i = _shape_edges(ui_edge_index[1])
    zd = jnp.zeros((NP, D), jnp.float32)

    degp = _degrees(ss_s, ss_d, ui_u, ui_i, zd)
    x0, x1, x2, acc, r = _tc_prep(user_emb, gate_W, gate_b, degp)

    p = _spmm_ss(x0, x1, x2, ss_s, ss_d, zd)
    acc, x0, x1, x2 = _tc_comb_ss(p, acc, r, need_x=True)
    p = _spmm_ss(x0, x1, x2, ss_s, ss_d, zd)
    (acc,) = _tc_comb_ss(p, acc, r, need_x=False)

    xu, xi, acc_ui = _tc_fuse(acc, r, item_emb, attn_W)

    p = _spmm_ui(xu, xi, ui_u, ui_i, zd)
    acc_ui, xu, xi = _tc_comb_ui(p, acc_ui, r, final=False)
    p = _spmm_ui(xu, xi, ui_u, ui_i, zd)
    return _tc_comb_ui(p, acc_ui, r, final=True)


# speedup vs baseline: 1.4365x; 1.4365x over previous
import functools

import jax
import jax.numpy as jnp
from jax import lax
from jax.experimental import pallas as pl
from jax.experimental.pallas import tpu as pltpu
from jax.experimental.pallas import tpu_sc as plsc

NU = 5000
NI = 5000
D = 128
E = 160000

NCORES = 2
NSUB = 16
NW = NCORES * NSUB
CH = 128
NCH = 40
EPAD = NW * NCH * CH
NP = NU + 8

_mesh = lambda: plsc.VectorSubcoreMesh(core_axis_name="c", subcore_axis_name="s")


def _zero_vmem(ref, rows, width):
    @pl.loop(0, rows)
    def _(i):
        for k in range(width // 16):
            ref[i, pl.ds(16 * k, 16)] = jnp.zeros((16,), jnp.float32)


def _fill_ones(ref, rows, width):
    @pl.loop(0, rows)
    def _(i):
        for k in range(width // 16):
            ref[i, pl.ds(16 * k, 16)] = jnp.ones((16,), jnp.float32)


def _rows_sweep(sub, fn):
    fn(pl.multiple_of(sub * 312, 8), 312)
    @pl.when(sub == 0)
    def _():
        fn(4992, 8)


def _copy_idx_row(src2d, j, dst1d):
    for i in range(CH // 16):
        dst1d[pl.ds(16 * i, 16)] = src2d[j, pl.ds(16 * i, 16)]


def _zero_rows(sub, acc, zhbm):
    _rows_sweep(sub, lambda off, n: pltpu.sync_copy(
        zhbm.at[pl.ds(off, n)], acc.at[pl.ds(off, n)]))


def _sc_degrees(ss_s, ss_d, ui_u, ui_i, zd, out_hbm,
                acc_a, acc_b, ix0, ix1, ix2, ix3, sidx_a, sidx_b,
                ones_v, sem_sa, sem_sb):
    core = lax.axis_index("c")
    sub = lax.axis_index("s")
    g = core * NSUB + sub

    _fill_ones(ones_v, CH, D)
    for edges, ix in zip((ss_s, ss_d, ui_u, ui_i), (ix0, ix1, ix2, ix3)):
        pltpu.sync_copy(edges.at[g], ix)

    for p, (ixa, ixb) in enumerate(((ix0, ix1), (ix2, ix3))):
        _zero_rows(sub, acc_a, zd)
        _zero_rows(sub, acc_b, zd)
        plsc.subcore_barrier()

        @pl.loop(0, NCH)
        def _(j):
            _copy_idx_row(ixa, j, sidx_a)
            _copy_idx_row(ixb, j, sidx_b)
            sa = pltpu.async_copy(ones_v, acc_a.at[sidx_a], sem_sa, add=True)
            sb = pltpu.async_copy(ones_v, acc_b.at[sidx_b], sem_sb, add=True)
            sa.wait()
            sb.wait()

        plsc.subcore_barrier()
        for base_off, acc in ((0, acc_a), (NU, acc_b)):
            _rows_sweep(sub, lambda off, n, a=acc, o=base_off, pp=p:
                        pltpu.sync_copy(
                            a.at[pl.ds(off, n)],
                            out_hbm.at[core, pp,
                                       pl.ds(pl.multiple_of(o + off, 8), n)]))


def _degrees(ss_s, ss_d, ui_u, ui_i, zd):
    fn = pl.kernel(
        _sc_degrees,
        out_type=jax.ShapeDtypeStruct((NCORES, 2, NU + NI, D), jnp.float32),
        mesh=_mesh(),
        scratch_types=[
            pltpu.VMEM_SHARED((NP, D), jnp.float32),
            pltpu.VMEM_SHARED((NP, D), jnp.float32),
            pltpu.VMEM((NCH, CH), jnp.int32),
            pltpu.VMEM((NCH, CH), jnp.int32),
            pltpu.VMEM((NCH, CH), jnp.int32),
            pltpu.VMEM((NCH, CH), jnp.int32),
            pltpu.VMEM((CH,), jnp.int32),
            pltpu.VMEM((CH,), jnp.int32),
            pltpu.VMEM((CH, D), jnp.float32),
            pltpu.SemaphoreType.DMA,
            pltpu.SemaphoreType.DMA,
        ],
    )
    return fn(ss_s, ss_d, ui_u, ui_i, zd)


_EPT = E // NW
_NF = _EPT // CH
_TL = _EPT - _NF * CH


def _sc_spmm_ss(x0, x1, x2, ss_s, ss_d, zd, out_hbm,
                acc, idx_s, idx_d, idx_st, idx_dt, rows_v, rows_t, sem):
    core = lax.axis_index("c")
    sub = lax.axis_index("s")
    g = core * NSUB + sub

    for c, x in enumerate((x0, x1, x2)):
        _zero_rows(sub, acc, zd)
        plsc.subcore_barrier()

        @pl.loop(0, _NF)
        def _(j):
            base = pl.multiple_of(g * _EPT + j * CH, 8)
            pltpu.sync_copy(ss_s.at[pl.ds(base, CH)], idx_s)
            pltpu.sync_copy(ss_d.at[pl.ds(base, CH)], idx_d)
            pltpu.async_copy(x.at[idx_s], rows_v, sem).wait()
            pltpu.sync_copy(rows_v, acc.at[idx_d], add=True)
        if _TL:
            base = pl.multiple_of(g * _EPT + _NF * CH, 8)
            pltpu.sync_copy(ss_s.at[pl.ds(base, _TL)], idx_st)
            pltpu.sync_copy(ss_d.at[pl.ds(base, _TL)], idx_dt)
            pltpu.async_copy(x.at[idx_st], rows_t, sem).wait()
            pltpu.sync_copy(rows_t, acc.at[idx_dt], add=True)

        plsc.subcore_barrier()
        _rows_sweep(sub, lambda off, n, ci=c: pltpu.sync_copy(
            acc.at[pl.ds(off, n)], out_hbm.at[core, ci, pl.ds(off, n)]))


def _spmm_ss(x0, x1, x2, ss_s, ss_d, zd):
    fn = pl.kernel(
        _sc_spmm_ss,
        out_type=jax.ShapeDtypeStruct((NCORES, 3, NU, D), jnp.float32),
        mesh=_mesh(),
        scratch_types=[
            pltpu.VMEM_SHARED((NP, D), jnp.float32),
            pltpu.VMEM((CH,), jnp.int32),
            pltpu.VMEM((CH,), jnp.int32),
            pltpu.VMEM((_TL,), jnp.int32),
            pltpu.VMEM((_TL,), jnp.int32),
            pltpu.VMEM((CH, D), jnp.float32),
            pltpu.VMEM((_TL, D), jnp.float32),
            pltpu.SemaphoreType.DMA,
        ],
    )
    return fn(x0, x1, x2, ss_s, ss_d, zd)


def _sc_spmm_ui(xu, xi, ui_u, ui_i, zd, out_hbm,
                accu, acci, idx_u, idx_i, sidx_u, sidx_i, rows_a, rows_b,
                sem_ga, sem_gb, sem_sa, sem_sb):
    core = lax.axis_index("c")
    sub = lax.axis_index("s")
    g = core * NSUB + sub

    pltpu.sync_copy(ui_u.at[g], idx_u)
    pltpu.sync_copy(ui_i.at[g], idx_i)
    for acc in (accu, acci):
        _zero_rows(sub, acc, zd)
    plsc.subcore_barrier()

    @pl.loop(0, NCH)
    def _(j):
        du = pltpu.async_copy(xu.at[idx_u.at[j]], rows_a, sem_ga)
        di = pltpu.async_copy(xi.at[idx_i.at[j]], rows_b, sem_gb)
        _copy_idx_row(idx_i, j, sidx_i)
        _copy_idx_row(idx_u, j, sidx_u)
        du.wait()
        sa = pltpu.async_copy(rows_a, acci.at[sidx_i], sem_sa, add=True)
        di.wait()
        sb = pltpu.async_copy(rows_b, accu.at[sidx_u], sem_sb, add=True)
        sa.wait()
        sb.wait()

    plsc.subcore_barrier()
    for base_off, acc in ((0, accu), (NU, acci)):
        _rows_sweep(sub, lambda off, n, a=acc, o=base_off: pltpu.sync_copy(
            a.at[pl.ds(off, n)],
            out_hbm.at[core, pl.ds(pl.multiple_of(o + off, 8), n)]))


def _spmm_ui(xu, xi, ui_u, ui_i, zd):
    fn = pl.kernel(
        _sc_spmm_ui,
        out_type=jax.ShapeDtypeStruct((NCORES, NU + NI, D), jnp.float32),
        mesh=_mesh(),
        scratch_types=[
            pltpu.VMEM_SHARED((NP, D), jnp.float32),
            pltpu.VMEM_SHARED((NP, D), jnp.float32),
            pltpu.VMEM((NCH, CH), jnp.int32),
            pltpu.VMEM((NCH, CH), jnp.int32),
            pltpu.VMEM((CH,), jnp.int32),
            pltpu.VMEM((CH,), jnp.int32),
            pltpu.VMEM((CH, D), jnp.float32),
            pltpu.VMEM((CH, D), jnp.float32),
            pltpu.SemaphoreType.DMA,
            pltpu.SemaphoreType.DMA,
            pltpu.SemaphoreType.DMA,
            pltpu.SemaphoreType.DMA,
        ],
    )
    return fn(xu, xi, ui_u, ui_i, zd)


def _tc_prep_body(u_ref, gw_ref, gb_ref, degp_ref,
                  x0_ref, x1_ref, x2_ref, acc_ref, r_ref):
    u = u_ref[...]
    degp = degp_ref[..., 0:1]
    dsum = degp[0] + degp[1]
    d = jnp.stack([dsum[0, 0:NU], dsum[0, NU:],
                   dsum[1, 0:NU], dsum[1, NU:]], axis=0)
    r = lax.rsqrt(d + 1.0)
    r_ref[...] = r
    rs = r[0]
    xr = (x0_ref, x1_ref, x2_ref)
    for c in range(3):
        gate = jax.nn.sigmoid(
            jnp.dot(u, gw_ref[c], preferred_element_type=jnp.float32)
            + gb_ref[c][None, :])
        cur = u * gate
        acc_ref[c] = cur
        xr[c][0:NU] = cur * rs
        xr[c][NU:] = jnp.zeros((NP - NU, D), jnp.float32)


def _tc_prep(user_emb, gate_W, gate_b, degp):
    sd = jax.ShapeDtypeStruct
    return pl.pallas_call(
        _tc_prep_body,
        out_shape=(sd((NP, D), jnp.float32), sd((NP, D), jnp.float32),
                   sd((NP, D), jnp.float32), sd((3, NU, D), jnp.float32),
                   sd((4, NU, 1), jnp.float32)),
    )(user_emb, gate_W, gate_b, degp)


def _tc_comb_ss_body(need_x, p_ref, accin_ref, r_ref, *outs):
    r = r_ref[...]
    cur = (p_ref[0] + p_ref[1]) * r[1][None]
    acc = accin_ref[...] + cur
    if need_x:
        acc_ref, x0_ref, x1_ref, x2_ref = outs
        rs = r[0]
        zpad = jnp.zeros((NP - NU, D), jnp.float32)
        for c, xref in enumerate((x0_ref, x1_ref, x2_ref)):
            xref[0:NU] = cur[c] * rs
            xref[NU:] = zpad
    else:
        (acc_ref,) = outs
    acc_ref[...] = acc


def _tc_comb_ss(partial, acc_in, r, need_x):
    sd = jax.ShapeDtypeStruct
    outs = (sd((3, NU, D), jnp.float32),)
    if need_x:
        outs = outs + (sd((NP, D), jnp.float32),) * 3
    return pl.pallas_call(
        functools.partial(_tc_comb_ss_body, need_x),
        out_shape=outs,
    )(partial, acc_in, r)


def _tc_fuse_body(acc_ref, r_ref, item_ref, attn_ref,
                  xu_ref, xi_ref, accui_ref):
    r = r_ref[...]
    item = item_ref[...]
    ch = acc_ref[...] * (1.0 / 3.0)
    t = jnp.tanh(ch)
    s = jnp.sum(t * attn_ref[...][None, None, :], axis=-1, keepdims=True)
    m = jnp.max(s, axis=0, keepdims=True)
    e = jnp.exp(s - m)
    w = e / jnp.sum(e, axis=0, keepdims=True)
    uf = jnp.sum(w * ch, axis=0)
    accui_ref[0:NU] = uf
    accui_ref[NU:] = item
    zpad = jnp.zeros((NP - NU, D), jnp.float32)
    xu_ref[0:NU] = uf * r[2]
    xu_ref[NU:] = zpad
    xi_ref[0:NI] = item * r[3]
    xi_ref[NI:] = zpad


def _tc_fuse(acc, r, item_emb, attn_W):
    sd = jax.ShapeDtypeStruct
    return pl.pallas_call(
        _tc_fuse_body,
        out_shape=(sd((NP, D), jnp.float32), sd((NP, D), jnp.float32),
                   sd((NU + NI, D), jnp.float32)),
    )(acc, r, item_emb, attn_W)


def _tc_comb_ui_body(final, p_ref, accin_ref, r_ref, *outs):
    r = r_ref[...]
    pp = p_ref[0] + p_ref[1]
    cur_u = pp[0:NU] * r[2]
    cur_i = pp[NU:] * r[3]
    acc_u = accin_ref[0:NU] + cur_u
    acc_i = accin_ref[NU:] + cur_i
    if final:
        (out_ref,) = outs
        out_ref[0:NU] = acc_u * (1.0 / 3.0)
        out_ref[NU:] = acc_i * (1.0 / 3.0)
    else:
        acc_ref, xu_ref, xi_ref = outs
        acc_ref[0:NU] = acc_u
        acc_ref[NU:] = acc_i
        zpad = jnp.zeros((NP - NU, D), jnp.float32)
        xu_ref[0:NU] = cur_u * r[2]
        xu_ref[NU:] = zpad
        xi_ref[0:NI] = cur_i * r[3]
        xi_ref[NI:] = zpad


def _tc_comb_ui(partial, acc_in, r, final):
    sd = jax.ShapeDtypeStruct
    if final:
        outs = sd((NU + NI, D), jnp.float32)
    else:
        outs = (sd((NU + NI, D), jnp.float32), sd((NP, D), jnp.float32),
                sd((NP, D), jnp.float32))
    return pl.pallas_call(
        functools.partial(_tc_comb_ui_body, final),
        out_shape=outs,
    )(partial, acc_in, r)


def kernel(user_emb, item_emb, gate_W, gate_b, attn_W,
           ui_edge_index, ss_edge_index):
    pad = jnp.full((EPAD - E,), NU, jnp.int32)
    def _shape_edges(v):
        return jnp.concatenate([v, pad]).reshape(NW, NCH, CH)
    ss_s = _shape_edges(ss_edge_index[0])
    ss_d = _shape_edges(ss_edge_index[1])
    ui_u = _shape_edges(ui_edge_index[0])
    ui_i = _shape_edges(ui_edge_index[1])
    zd = jnp.zeros((NP, D), jnp.float32)

    degp = _degrees(ss_s, ss_d, ui_u, ui_i, zd)
    x0, x1, x2, acc, r = _tc_prep(user_emb, gate_W, gate_b, degp)

    ss_s1, ss_d1 = ss_edge_index[0], ss_edge_index[1]

    p = _spmm_ss(x0, x1, x2, ss_s1, ss_d1, zd)
    acc, x0, x1, x2 = _tc_comb_ss(p, acc, r, need_x=True)
    p = _spmm_ss(x0, x1, x2, ss_s1, ss_d1, zd)
    (acc,) = _tc_comb_ss(p, acc, r, need_x=False)

    xu, xi, acc_ui = _tc_fuse(acc, r, item_emb, attn_W)

    p = _spmm_ui(xu, xi, ui_u, ui_i, zd)
    acc_ui, xu, xi = _tc_comb_ui(p, acc_ui, r, final=False)
    p = _spmm_ui(xu, xi, ui_u, ui_i, zd)
    return _tc_comb_ui(p, acc_ui, r, final=True)
